# baseline (speedup 1.0000x reference)
import jax
import jax.numpy as jnp
from jax.experimental import pallas as pl
from jax.experimental.pallas import tpu as pltpu

_ROWS_PER_STEP = 10


def _add_pe_body(s_ref, x_ref, pe_ref, o_ref):
    i = pl.program_id(0)
    for r in range(_ROWS_PER_STEP):
        row = s_ref[i * _ROWS_PER_STEP + r]
        o_ref[r, :, :] = x_ref[r, :, :] + pe_ref[row, :, :]


def kernel(x, pe, source_encoding):
    var_num, batch, d_model = x.shape
    max_len = pe.shape[1]
    pe3d = pe.reshape(max_len, 1, d_model)
    grid = (var_num // _ROWS_PER_STEP,)
    return pl.pallas_call(
        _add_pe_body,
        grid_spec=pltpu.PrefetchScalarGridSpec(
            num_scalar_prefetch=1,
            grid=grid,
            in_specs=[
                pl.BlockSpec((_ROWS_PER_STEP, batch, d_model),
                             lambda i, s: (i, 0, 0)),
                pl.BlockSpec((max_len, 1, d_model), lambda i, s: (0, 0, 0)),
            ],
            out_specs=pl.BlockSpec((_ROWS_PER_STEP, batch, d_model),
                                   lambda i, s: (i, 0, 0)),
        ),
        out_shape=jax.ShapeDtypeStruct(x.shape, x.dtype),
    )(source_encoding, x, pe3d)


# speedup vs baseline: 1.0216x; 1.0216x over previous
import jax
import jax.numpy as jnp
from jax.experimental import pallas as pl
from jax.experimental.pallas import tpu as pltpu

_ROWS_PER_STEP = 25


def _add_pe_body(s_ref, x_ref, pe_ref, o_ref):
    i = pl.program_id(0)
    for r in range(_ROWS_PER_STEP):
        row = s_ref[i * _ROWS_PER_STEP + r]
        o_ref[r, :, :] = x_ref[r, :, :] + pe_ref[row, :, :]


def kernel(x, pe, source_encoding):
    var_num, batch, d_model = x.shape
    max_len = pe.shape[1]
    pe3d = pe.reshape(max_len, 1, d_model)
    grid = (var_num // _ROWS_PER_STEP,)
    return pl.pallas_call(
        _add_pe_body,
        grid_spec=pltpu.PrefetchScalarGridSpec(
            num_scalar_prefetch=1,
            grid=grid,
            in_specs=[
                pl.BlockSpec((_ROWS_PER_STEP, batch, d_model),
                             lambda i, s: (i, 0, 0)),
                pl.BlockSpec((max_len, 1, d_model), lambda i, s: (0, 0, 0)),
            ],
            out_specs=pl.BlockSpec((_ROWS_PER_STEP, batch, d_model),
                                   lambda i, s: (i, 0, 0)),
        ),
        out_shape=jax.ShapeDtypeStruct(x.shape, x.dtype),
    )(source_encoding, x, pe3d)
